# Initial kernel scaffold; baseline (speedup 1.0000x reference)
import jax
import jax.numpy as jnp
from jax.experimental import pallas as pl


def kernel(atom_type, chirality, bond_type, bond_dir, edge_index, node2graph, atom_emb1, atom_emb2, edge_emb1, edge_emb2, W1, b1, W2, b2, bn_g, bn_b, fW1, fb1, fg1, fbt1, fW2, fb2, fg2, fbt2, fW3, fb3):
    raise NotImplementedError("write your pallas kernel here")


# speedup vs baseline: 3.4429x; 3.4429x over previous
import functools

import jax
import jax.numpy as jnp
from jax import lax
from jax.experimental import pallas as pl
from jax.experimental.pallas import tpu as pltpu
from jax.experimental.pallas import tpu_sc as plsc

N = 10000
E = 160000
G = 256
EMB = 300
NL = 5
W = 128
NBAND = 3
KW = 128
NH = N // 2
NRL = 5120
RPT = NRL // 16
DUMMY = 5000
CW = 64
EPAD = 163840
ECH = EPAD // 128
RPS = ECH // 16
CAP = RPS * 128
NCH = CAP // CW
NB = 10
BLK = N // NB

_f32 = jnp.float32
_i32 = jnp.int32


@functools.lru_cache(maxsize=None)
def _sc_mesh():
    return plsc.VectorSubcoreMesh(core_axis_name="c", subcore_axis_name="s")


def _zero_acc(zrow, buf, acc, base):
    pltpu.sync_copy(zrow, buf)
    for kk in range(RPT // CW):
        pltpu.sync_copy(buf, acc.at[pl.ds(base + kk * CW, CW)])


def _writeback(acc, buf, out_c, base):
    for kk in range(RPT // CW):
        sl = pl.ds(base + kk * CW, CW)
        pltpu.sync_copy(acc.at[sl], buf)
        pltpu.sync_copy(buf, out_c.at[sl])


def _stage_idx(src1d, j, idx_cur):
    for t in range(CW // 16):
        idx_cur[pl.ds(16 * t, 16)] = src1d[pl.ds(j * CW + 16 * t, 16)]


def _route_body(src2d, dst2d, k2d, eye, zrowk,
                csrc_o, cdst_o, ccnt_o, counts_o,
                srcv, dstv, kvv, c1, d1, k1, buf, cntv, idx_cur, cacc,
                gsem, ssem):
    c = lax.axis_index("c")
    s = lax.axis_index("s")
    lo = c * NH
    pltpu.sync_copy(src2d.at[pl.ds(s * RPS, RPS)], srcv)
    pltpu.sync_copy(dst2d.at[pl.ds(s * RPS, RPS)], dstv)
    pltpu.sync_copy(k2d.at[pl.ds(s * RPS, RPS)], kvv)
    zi = jnp.zeros((16,), _i32)
    dum = jnp.full((16,), DUMMY, _i32)

    def pre(j, carry):
        sl = pl.ds(j * 16, 16)
        c1[sl] = zi
        d1[sl] = dum
        k1[sl] = zi
        return carry

    lax.fori_loop(0, CAP // 16, pre, 0)

    def cstep(j, pos):
        i = j // 8
        jj = j % 8
        sl = pl.ds(jj * 16, 16)
        d = dstv[i, sl]
        sv = srcv[i, sl]
        kv = kvv[i, sl]
        ld = d - lo
        m = (ld >= 0) & (ld < NH)
        cs = plsc.cumsum(m.astype(_i32))
        idx = jnp.where(m, pos + cs - 1, CAP + 8)
        plsc.store_scatter(c1, [idx], sv, mask=m)
        plsc.store_scatter(d1, [idx], ld, mask=m)
        plsc.store_scatter(k1, [idx], kv, mask=m)
        return pos + cs[15]

    cnt = lax.fori_loop(0, RPS * 8, cstep, 0)

    pltpu.sync_copy(c1.at[pl.ds(0, CAP)], csrc_o.at[c, s])
    pltpu.sync_copy(d1.at[pl.ds(0, CAP)], cdst_o.at[c, s])
    cntv[...] = jnp.broadcast_to(cnt, (16,))
    pltpu.sync_copy(cntv, ccnt_o.at[c, s])

    base = s * RPT
    _zero_acc(zrowk, buf, cacc, base)
    plsc.subcore_barrier()
    nch = (cnt + (CW - 1)) // CW

    def hstep(j, carry):
        pltpu.async_copy(eye.at[k1.at[pl.ds(j * CW, CW)]], buf, gsem).wait()
        _stage_idx(d1, j, idx_cur)
        pltpu.async_copy(buf, cacc.at[idx_cur], ssem, add=True).wait()
        return carry

    lax.fori_loop(0, nch, hstep, 0)
    plsc.subcore_barrier()
    _writeback(cacc, buf, counts_o.at[c], base)


@functools.lru_cache(maxsize=None)
def _route_call():
    return pl.kernel(
        _route_body,
        out_type=[jax.ShapeDtypeStruct((2, 16, CAP), _i32),
                  jax.ShapeDtypeStruct((2, 16, CAP), _i32),
                  jax.ShapeDtypeStruct((2, 16, 16), _i32),
                  jax.ShapeDtypeStruct((2, NRL, KW), _f32)],
        mesh=_sc_mesh(),
        compiler_params=pltpu.CompilerParams(needs_layout_passes=False),
        scratch_types=[
            pltpu.VMEM((RPS, 128), _i32),
            pltpu.VMEM((RPS, 128), _i32),
            pltpu.VMEM((RPS, 128), _i32),
            pltpu.VMEM((CAP + 16,), _i32),
            pltpu.VMEM((CAP + 16,), _i32),
            pltpu.VMEM((CAP + 16,), _i32),
            pltpu.VMEM((CW, KW), _f32),
            pltpu.VMEM((16,), _i32),
            pltpu.VMEM((CW,), _i32),
            pltpu.VMEM_SHARED((NRL, KW), _f32),
            pltpu.SemaphoreType.DMA,
            pltpu.SemaphoreType.DMA,
        ],
    )


def _route(src2d, dst2d, k2d, eye, zrowk):
    return _route_call()(src2d, dst2d, k2d, eye, zrowk)


def _spmm_body(h, csrc, cdst, ccnt, zrow, agg_o,
               csv, cdv, cntv, buf, idx_cur, acc, gsem, ssem):
    c = lax.axis_index("c")
    s = lax.axis_index("s")
    pltpu.sync_copy(ccnt.at[c, s], cntv)
    cnt = cntv[...][0]
    nch = (cnt + (CW - 1)) // CW
    base = s * RPT
    _zero_acc(zrow, buf, acc, base)
    pltpu.sync_copy(csrc.at[c, s], csv)
    pltpu.sync_copy(cdst.at[c, s], cdv)
    plsc.subcore_barrier()

    def step(j, carry):
        pltpu.async_copy(h.at[csv.at[pl.ds(j * CW, CW)]], buf, gsem).wait()
        _stage_idx(cdv, j, idx_cur)
        pltpu.async_copy(buf, acc.at[idx_cur], ssem, add=True).wait()
        return carry

    lax.fori_loop(0, nch, step, 0)
    plsc.subcore_barrier()
    _writeback(acc, buf, agg_o.at[c], base)


@functools.lru_cache(maxsize=None)
def _spmm_call():
    return pl.kernel(
        _spmm_body,
        out_type=jax.ShapeDtypeStruct((2, NRL, W), _f32),
        mesh=_sc_mesh(),
        compiler_params=pltpu.CompilerParams(needs_layout_passes=False),
        scratch_types=[
            pltpu.VMEM((CAP,), _i32),
            pltpu.VMEM((CAP,), _i32),
            pltpu.VMEM((16,), _i32),
            pltpu.VMEM((CW, W), _f32),
            pltpu.VMEM((CW,), _i32),
            pltpu.VMEM_SHARED((NRL, W), _f32),
            pltpu.SemaphoreType.DMA,
            pltpu.SemaphoreType.DMA,
        ],
    )


def _spmm(h, csrc, cdst, ccnt, zrow):
    return _spmm_call()(h, csrc, cdst, ccnt, zrow)


def _rne_bf16(x):
    u = lax.bitcast_convert_type(x, jnp.uint32)
    r = (u + jnp.uint32(0x7FFF) + ((u >> jnp.uint32(16)) & jnp.uint32(1))
         ) & jnp.uint32(0xFFFF0000)
    return lax.bitcast_convert_type(r, _f32)


def _emb_kernel(at_ref, ch_ref, e1_ref, e2_ref, h0_ref, h1_ref, h2_ref):
    at = at_ref[0, 0, :]
    ch = ch_ref[0, 0, :]
    oh1 = (at[:, None] == lax.broadcasted_iota(_i32, (BLK, 120), 1)
           ).astype(_f32)
    oh2 = (ch[:, None] == lax.broadcasted_iota(_i32, (BLK, 8), 1)
           ).astype(_f32)
    h = jnp.dot(oh1, e1_ref[...], preferred_element_type=_f32, precision=lax.Precision.HIGHEST)
    h = h + jnp.dot(oh2, e2_ref[...], preferred_element_type=_f32, precision=lax.Precision.HIGHEST)
    h0_ref[...] = h[:, :W]
    h1_ref[...] = h[:, W:2 * W]
    h2_ref[...] = h[:, 2 * W:]


def _layer_a_kernel(a0_ref, a1_ref, a2_ref, cnt_ref, tl_ref, w1_ref, b1_ref,
                    w2_ref, b2_ref, u_ref, st_ref):
    i = pl.program_id(0)
    x = jnp.concatenate([a0_ref[0], a1_ref[0], a2_ref[0]], axis=1)[:, :EMB]
    x = x + jnp.dot(cnt_ref[0], tl_ref[...], preferred_element_type=_f32,
                    precision=lax.Precision.HIGHEST)
    hp = lax.Precision.HIGHEST
    y = jnp.maximum(
        jnp.dot(_rne_bf16(x), w1_ref[...], preferred_element_type=_f32,
                precision=hp) + b1_ref[0:1, :], 0.0)
    u = jnp.dot(_rne_bf16(y), w2_ref[...], preferred_element_type=_f32,
                precision=hp) + b2_ref[0:1, :]
    u_ref[...] = u

    @pl.when(i == 0)
    def _():
        st_ref[...] = jnp.zeros_like(st_ref)

    su = jnp.sum(u, axis=0)
    sq = jnp.sum(u * u, axis=0)
    st_ref[...] = st_ref[...] + jnp.concatenate(
        [su[None, :], sq[None, :], jnp.zeros((6, EMB), _f32)], axis=0)


def _layer_b_kernel(relu, u_ref, st_ref, g_ref, b_ref,
                    h0_ref, h1_ref, h2_ref):
    mu = st_ref[0:1, :] * (1.0 / N)
    var = st_ref[1:2, :] * (1.0 / N) - mu * mu
    inv = lax.rsqrt(var + 1e-5)
    scale = g_ref[0:1, :] * inv
    shift = b_ref[0:1, :] - mu * scale
    z = u_ref[...] * scale + shift
    if relu:
        z = jnp.maximum(z, 0.0)
    h0_ref[...] = z[:, :W]
    h1_ref[...] = z[:, W:2 * W]
    h2_ref[...] = jnp.concatenate(
        [z[:, 2 * W:], jnp.zeros((BLK, NBAND * W - EMB), _f32)], axis=1)


def _pool_kernel(*refs):
    n2g_ref = refs[0]
    pieces = refs[1:19]
    outs = refs[19:37]
    gcnt_ref = refs[37]
    i = pl.program_id(0)

    @pl.when(i == 0)
    def _():
        for o in outs:
            o[...] = jnp.zeros_like(o)
        gcnt_ref[...] = jnp.zeros_like(gcnt_ref)

    n2g = n2g_ref[0, 0, :]
    p = (n2g[:, None] == lax.broadcasted_iota(_i32, (BLK, G), 1)
         ).astype(_f32)
    for pc, o in zip(pieces, outs):
        o[...] = o[...] + lax.dot_general(
            p, pc[...], (((0,), (0,)), ((), ())),
            preferred_element_type=_f32, precision=lax.Precision.HIGHEST)
    cnt = jnp.sum(p, axis=0)
    gcnt_ref[...] = gcnt_ref[...] + jnp.concatenate(
        [cnt[None, :], jnp.zeros((7, G), _f32)], axis=0)


def _head_kernel(*refs):
    pieces = refs[0:18]
    (gcnt_ref, fw1_ref, fb1_ref, fg1_ref, fbt1_ref,
     fw2_ref, fb2_ref, fg2_ref, fbt2_ref, fw3_ref, fb3_ref, out_ref) = refs[18:]
    cnt = jnp.clip(gcnt_ref[0:1, :], 1.0, None)
    parts = []
    for l in range(6):
        parts.append(pieces[3 * l][...])
        parts.append(pieces[3 * l + 1][...])
        parts.append(pieces[3 * l + 2][...][:, :EMB - 2 * W])
    x = jnp.concatenate(parts, axis=1)
    x = x * (1.0 / cnt).T

    def bn(v, g, b):
        mu = jnp.mean(v, axis=0, keepdims=True)
        var = jnp.mean((v - mu) * (v - mu), axis=0, keepdims=True)
        return (v - mu) * lax.rsqrt(var + 1e-5) * g + b

    hp = lax.Precision.HIGHEST
    a = jnp.maximum(
        jnp.dot(_rne_bf16(x), fw1_ref[...], preferred_element_type=_f32,
                precision=hp) + fb1_ref[0:1, :], 0.0)
    a = bn(a, fg1_ref[0:1, :], fbt1_ref[0:1, :])
    a = jnp.maximum(
        jnp.dot(_rne_bf16(a), fw2_ref[...], preferred_element_type=_f32,
                precision=hp) + fb2_ref[0:1, :], 0.0)
    a = bn(a, fg2_ref[0:1, :], fbt2_ref[0:1, :])
    out_ref[...] = (jnp.dot(_rne_bf16(a), fw3_ref[...],
                            preferred_element_type=_f32, precision=hp)
                    + fb3_ref[0:1, :])


def _blk_spec(w):
    return pl.BlockSpec((BLK, w), lambda i: (i, 0))


def _loc_spec(w):
    return pl.BlockSpec((1, BLK, w), lambda i: (i // 5, i % 5, 0))


def _full_spec(shape):
    return pl.BlockSpec(shape, lambda i: tuple(0 for _ in shape))


def _idx_spec():
    return pl.BlockSpec((1, 1, BLK), lambda i: (i, 0, 0))


_emb_call = pl.pallas_call(
    _emb_kernel,
    grid=(NB,),
    in_specs=[_idx_spec(), _idx_spec(), _full_spec((120, NBAND * W)),
              _full_spec((8, NBAND * W))],
    out_specs=[_blk_spec(W)] * NBAND,
    out_shape=[jax.ShapeDtypeStruct((N, W), _f32)] * NBAND,
)

_layer_a_call = pl.pallas_call(
    _layer_a_kernel,
    grid=(NB,),
    in_specs=[_loc_spec(W)] * NBAND + [_loc_spec(KW),
              _full_spec((KW, EMB)), _full_spec((EMB, 2 * EMB)),
              _full_spec((8, 2 * EMB)),
              _full_spec((2 * EMB, EMB)), _full_spec((8, EMB))],
    out_specs=[_blk_spec(EMB), _full_spec((8, EMB))],
    out_shape=[jax.ShapeDtypeStruct((N, EMB), _f32),
               jax.ShapeDtypeStruct((8, EMB), _f32)],
)


def _make_layer_b(relu):
    return pl.pallas_call(
        functools.partial(_layer_b_kernel, relu),
        grid=(NB,),
        in_specs=[_blk_spec(EMB), _full_spec((8, EMB)),
                  _full_spec((8, EMB)), _full_spec((8, EMB))],
        out_specs=[_blk_spec(W)] * NBAND,
        out_shape=[jax.ShapeDtypeStruct((N, W), _f32)] * NBAND,
    )


_layer_b_relu = _make_layer_b(True)
_layer_b_last = _make_layer_b(False)

_pool_call = pl.pallas_call(
    _pool_kernel,
    grid=(NB,),
    in_specs=[_idx_spec()] + [_blk_spec(W)] * 18,
    out_specs=[_full_spec((G, W))] * 18 + [_full_spec((8, G))],
    out_shape=[jax.ShapeDtypeStruct((G, W), _f32)] * 18
    + [jax.ShapeDtypeStruct((8, G), _f32)],
)

_head_call = pl.pallas_call(
    _head_kernel,
    grid=(1,),
    in_specs=[_full_spec((G, W))] * 18
    + [_full_spec((8, G)), _full_spec((6 * EMB, 512)), _full_spec((8, 512)),
       _full_spec((8, 512)), _full_spec((8, 512)), _full_spec((512, 512)),
       _full_spec((8, 512)), _full_spec((8, 512)), _full_spec((8, 512)),
       _full_spec((512, 128)), _full_spec((8, 128))],
    out_specs=_full_spec((G, 128)),
    out_shape=jax.ShapeDtypeStruct((G, 128), _f32),
)


def _row8(v):
    return jnp.broadcast_to(v[None, :], (8, v.shape[0])).astype(_f32)


def _pad_cols(w, width):
    return jnp.concatenate(
        [w.astype(_f32), jnp.zeros((w.shape[0], width - w.shape[1]), _f32)],
        axis=1)


def kernel(atom_type, chirality, bond_type, bond_dir, edge_index, node2graph,
           atom_emb1, atom_emb2, edge_emb1, edge_emb2, W1, b1, W2, b2,
           bn_g, bn_b, fW1, fb1, fg1, fbt1, fW2, fb2, fg2, fbt2, fW3, fb3):
    src = edge_index[0].astype(_i32)
    dst = edge_index[1].astype(_i32)
    kidx = bond_type.astype(_i32) * 3 + bond_dir.astype(_i32)
    pad = EPAD - E
    src2d = jnp.concatenate([src, jnp.zeros((pad,), _i32)]).reshape(ECH, 128)
    dst2d = jnp.concatenate([dst, jnp.full((pad,), N, _i32)]).reshape(ECH, 128)
    k2d = jnp.concatenate([kidx, jnp.zeros((pad,), _i32)]).reshape(ECH, 128)
    zrow = jnp.zeros((CW, W), _f32)
    eye = jnp.eye(24, KW, dtype=_f32)

    at3 = atom_type.astype(_i32).reshape(NB, 1, BLK)
    ch3 = chirality.astype(_i32).reshape(NB, 1, BLK)
    n2g3 = node2graph.astype(_i32).reshape(NB, 1, BLK)

    e1p = _pad_cols(atom_emb1, NBAND * W)
    e2p = _pad_cols(jnp.concatenate(
        [atom_emb2.astype(_f32), jnp.zeros((5, EMB), _f32)], axis=0),
        NBAND * W)

    csrc, cdst, ccnt, counts2 = _route(src2d, dst2d, k2d, eye, zrow)
    hbands = list(_emb_call(at3, ch3, e1p, e2p))

    pieces = list(hbands)
    for l in range(NL):
        w1l = W1[l].astype(_f32)
        t_l = (jnp.repeat(edge_emb1[l].astype(_f32), 3, axis=0)
               + jnp.tile(edge_emb2[l].astype(_f32), (6, 1)))
        tlp = jnp.concatenate([t_l, jnp.zeros((KW - 18, EMB), _f32)], axis=0)

        aggs = [_spmm(hb, csrc, cdst, ccnt, zrow) for hb in hbands]
        u, st = _layer_a_call(*aggs, counts2, tlp,
                              w1l.astype(jnp.bfloat16).astype(_f32),
                              _row8(b1[l]),
                              W2[l].astype(jnp.bfloat16).astype(_f32),
                              _row8(b2[l]))
        lb = _layer_b_relu if l < NL - 1 else _layer_b_last
        hbands = list(lb(u, st, _row8(bn_g[l]), _row8(bn_b[l])))
        pieces += hbands

    pool_out = _pool_call(n2g3, *pieces)
    gsums, gcnt = pool_out[:18], pool_out[18]
    b16f = lambda w: w.astype(jnp.bfloat16).astype(_f32)
    out = _head_call(*gsums, gcnt, b16f(fW1), _row8(fb1),
                     _row8(fg1), _row8(fbt1), b16f(fW2),
                     _row8(fb2), _row8(fg2), _row8(fbt2),
                     b16f(fW3), _row8(fb3))
    return out
